# baseline (speedup 1.0000x reference)
import functools

import jax
import jax.numpy as jnp
from jax import lax
from jax.experimental import pallas as pl
from jax.experimental.pallas import tpu as pltpu
from jax.experimental.pallas import tpu_sc as plsc

B = 16384
D = 32
L = 16
NC, NS = 2, 16
NW = NC * NS
BPW = B // NW

_mesh = plsc.VectorSubcoreMesh(core_axis_name="c", subcore_axis_name="s")


@functools.partial(
    pl.kernel,
    mesh=_mesh,
    out_type=jax.ShapeDtypeStruct((B,), jnp.float32),
    scratch_types=[
        pltpu.VMEM((BPW,), jnp.int32),
        pltpu.VMEM((BPW,), jnp.int32),
        pltpu.VMEM((BPW, D), jnp.float32),
        pltpu.VMEM((BPW, D), jnp.float32),
        pltpu.VMEM((BPW,), jnp.float32),
        pltpu.SemaphoreType.DMA,
        pltpu.SemaphoreType.DMA,
    ],
    compiler_params=pltpu.CompilerParams(
        needs_layout_passes=False, use_tc_tiling_on_sc=False
    ),
)
def _mf_sc(uid_hbm, iid_hbm, ut_hbm, it_hbm, out_hbm,
           uidx, iidx, urows, irows, outv, sem_u, sem_i):
    wid = lax.axis_index("s") * NC + lax.axis_index("c")
    base = wid * BPW
    pltpu.sync_copy(uid_hbm.at[pl.ds(base, BPW)], uidx)
    pltpu.sync_copy(iid_hbm.at[pl.ds(base, BPW)], iidx)
    cu = pltpu.async_copy(ut_hbm.at[uidx], urows, sem_u)
    ci = pltpu.async_copy(it_hbm.at[iidx], irows, sem_i)
    cu.wait()
    ci.wait()

    def body(c, carry):
        rows = c * L + lax.iota(jnp.int32, L)
        acc = jnp.zeros((L,), jnp.float32)
        for dd in range(D):
            col = jnp.full((L,), dd, jnp.int32)
            uv = plsc.load_gather(urows, [rows, col])
            iv = plsc.load_gather(irows, [rows, col])
            acc = acc + uv * iv
        outv[pl.ds(c * L, L)] = acc
        return carry

    lax.fori_loop(0, BPW // L, body, 0, unroll=False)
    pltpu.sync_copy(outv, out_hbm.at[pl.ds(base, BPW)])


def kernel(user_id, item_id, user_table, item_table):
    return _mf_sc(user_id.astype(jnp.int32), item_id.astype(jnp.int32),
                  user_table, item_table)


# speedup vs baseline: 3.5927x; 3.5927x over previous
import functools

import jax
import jax.numpy as jnp
from jax import lax
from jax.experimental import pallas as pl
from jax.experimental.pallas import tpu as pltpu
from jax.experimental.pallas import tpu_sc as plsc

N_ROWS = 1000000
B = 16384
D = 32
BLK = 128
W = 8
L = 16
NC, NS = 2, 16
NW = NC * NS
BPW = B // NW
PAD = L

_mesh = plsc.VectorSubcoreMesh(core_axis_name="c", subcore_axis_name="s")


@functools.partial(
    pl.kernel,
    mesh=_mesh,
    out_type=jax.ShapeDtypeStruct((B,), jnp.float32),
    scratch_types=[
        pltpu.VMEM((BPW + PAD,), jnp.int32),
        pltpu.VMEM((BPW + PAD,), jnp.int32),
        pltpu.VMEM((W, D, BLK), jnp.float32),
        pltpu.VMEM((W, D, BLK), jnp.float32),
        pltpu.VMEM((BPW + PAD,), jnp.float32),
        pltpu.SemaphoreType.DMA,
        pltpu.SemaphoreType.DMA,
    ],
    compiler_params=pltpu.CompilerParams(
        needs_layout_passes=False, disable_bounds_checks=True
    ),
)
def _mf_sc(uid_hbm, iid_hbm, ut_hbm, it_hbm, out_hbm,
           uidx, iidx, ublk, iblk, outv, sem_u, sem_i):
    wid = lax.axis_index("s") * NC + lax.axis_index("c")
    base = wid * BPW
    pltpu.sync_copy(uid_hbm.at[pl.ds(base, BPW)], uidx.at[pl.ds(0, BPW)])
    pltpu.sync_copy(iid_hbm.at[pl.ds(base, BPW)], iidx.at[pl.ds(0, BPW)])

    lanes = lax.iota(jnp.int32, L)
    t_sel = lax.bitwise_and(lanes, jnp.int32(W - 1))
    lo_mask = lanes < W

    def wave(w, carry):
        uv = uidx[pl.ds(w * W, L)]
        iv = iidx[pl.ds(w * W, L)]
        copies = []
        for t in range(W):
            ub = lax.shift_right_logical(uv[t], 7)
            ib = lax.shift_right_logical(iv[t], 7)
            copies.append(pltpu.async_copy(
                ut_hbm.at[:, pl.ds(pl.multiple_of(ub * BLK, BLK), BLK)],
                ublk.at[t], sem_u))
            copies.append(pltpu.async_copy(
                it_hbm.at[:, pl.ds(pl.multiple_of(ib * BLK, BLK), BLK)],
                iblk.at[t], sem_i))
        for c in copies:
            c.wait()

        um = lax.bitwise_and(uv, jnp.int32(BLK - 1))
        im = lax.bitwise_and(iv, jnp.int32(BLK - 1))
        acc = jnp.zeros((L,), jnp.float32)
        for dd in range(D):
            dv = jnp.full((L,), dd, jnp.int32)
            ue = plsc.load_gather(ublk, [t_sel, dv, um])
            ie = plsc.load_gather(iblk, [t_sel, dv, im])
            acc = acc + ue * ie
        plsc.store_compressed(outv.at[pl.ds(w * W, L)], acc, mask=lo_mask)
        return carry

    lax.fori_loop(0, BPW // W, wave, 0, unroll=False)
    pltpu.sync_copy(outv.at[pl.ds(0, BPW)], out_hbm.at[pl.ds(base, BPW)])


def kernel(user_id, item_id, user_table, item_table):
    return _mf_sc(user_id.astype(jnp.int32), item_id.astype(jnp.int32),
                  user_table.T, item_table.T)


# speedup vs baseline: 4.0980x; 1.1407x over previous
import functools

import jax
import jax.numpy as jnp
from jax import lax
from jax.experimental import pallas as pl
from jax.experimental.pallas import tpu as pltpu
from jax.experimental.pallas import tpu_sc as plsc

N_ROWS = 1000000
B = 16384
D = 32
BLK = 128
W = 4
NBUF = 2
L = 16
NC, NS = 2, 16
NW = NC * NS
BPW = B // NW
NWAVE = BPW // W
PAD = L

_mesh = plsc.VectorSubcoreMesh(core_axis_name="c", subcore_axis_name="s")


@functools.partial(
    pl.kernel,
    mesh=_mesh,
    out_type=jax.ShapeDtypeStruct((B,), jnp.float32),
    scratch_types=[
        pltpu.VMEM((BPW + PAD,), jnp.int32),
        pltpu.VMEM((BPW + PAD,), jnp.int32),
        pltpu.VMEM((NBUF, W, D, BLK), jnp.float32),
        pltpu.VMEM((NBUF, W, D, BLK), jnp.float32),
        pltpu.VMEM((BPW + PAD,), jnp.float32),
        pltpu.SemaphoreType.DMA,
        pltpu.SemaphoreType.DMA,
    ],
    compiler_params=pltpu.CompilerParams(
        needs_layout_passes=False, disable_bounds_checks=True
    ),
)
def _mf_sc(uid_hbm, iid_hbm, ut_hbm, it_hbm, out_hbm,
           uidx, iidx, ublk, iblk, outv, sem_u, sem_i):
    wid = lax.axis_index("s") * NC + lax.axis_index("c")
    base = wid * BPW
    pltpu.sync_copy(uid_hbm.at[pl.ds(base, BPW)], uidx.at[pl.ds(0, BPW)])
    pltpu.sync_copy(iid_hbm.at[pl.ds(base, BPW)], iidx.at[pl.ds(0, BPW)])

    lanes = lax.iota(jnp.int32, L)
    t_sel = lax.bitwise_and(lanes, jnp.int32(W - 1))
    lo_mask = lanes < W

    def fire(w):
        buf = lax.rem(w, NBUF)
        uv = uidx[pl.ds(w * W, L)]
        iv = iidx[pl.ds(w * W, L)]
        for t in range(W):
            ub = lax.shift_right_logical(uv[t], 7)
            ib = lax.shift_right_logical(iv[t], 7)
            pltpu.async_copy(
                ut_hbm.at[:, pl.ds(pl.multiple_of(ub * BLK, BLK), BLK)],
                ublk.at[buf, t], sem_u)
            pltpu.async_copy(
                it_hbm.at[:, pl.ds(pl.multiple_of(ib * BLK, BLK), BLK)],
                iblk.at[buf, t], sem_i)

    fire(jnp.int32(0))

    def wave(w, carry):
        @pl.when(w + 1 < NWAVE)
        def _():
            fire(w + 1)

        buf = lax.rem(w, NBUF)
        for t in range(W):
            pltpu.make_async_copy(
                ut_hbm.at[:, pl.ds(0, BLK)], ublk.at[buf, t], sem_u).wait()
            pltpu.make_async_copy(
                it_hbm.at[:, pl.ds(0, BLK)], iblk.at[buf, t], sem_i).wait()

        uv = uidx[pl.ds(w * W, L)]
        iv = iidx[pl.ds(w * W, L)]
        um = lax.bitwise_and(uv, jnp.int32(BLK - 1))
        im = lax.bitwise_and(iv, jnp.int32(BLK - 1))
        ub = ublk.at[buf]
        ib = iblk.at[buf]
        acc = jnp.zeros((L,), jnp.float32)
        for dd in range(D):
            dv = jnp.full((L,), dd, jnp.int32)
            ue = plsc.load_gather(ub, [t_sel, dv, um])
            ie = plsc.load_gather(ib, [t_sel, dv, im])
            acc = acc + ue * ie
        plsc.store_compressed(outv.at[pl.ds(w * W, L)], acc, mask=lo_mask)
        return carry

    lax.fori_loop(0, NWAVE, wave, 0, unroll=False)
    pltpu.sync_copy(outv.at[pl.ds(0, BPW)], out_hbm.at[pl.ds(base, BPW)])


def kernel(user_id, item_id, user_table, item_table):
    return _mf_sc(user_id.astype(jnp.int32), item_id.astype(jnp.int32),
                  user_table.T, item_table.T)


# speedup vs baseline: 4.4185x; 1.0782x over previous
import functools

import jax
import jax.numpy as jnp
from jax import lax
from jax.experimental import pallas as pl
from jax.experimental.pallas import tpu as pltpu
from jax.experimental.pallas import tpu_sc as plsc

N_ROWS = 1000000
B = 16384
D = 32
BLK = 128
W = 4
NBUF = 3
L = 16
NC, NS = 2, 16
NW = NC * NS
BPW = B // NW
NWAVE = BPW // W
PAD = L

_mesh = plsc.VectorSubcoreMesh(core_axis_name="c", subcore_axis_name="s")


@functools.partial(
    pl.kernel,
    mesh=_mesh,
    out_type=jax.ShapeDtypeStruct((B,), jnp.float32),
    scratch_types=[
        pltpu.VMEM((BPW + PAD,), jnp.int32),
        pltpu.VMEM((BPW + PAD,), jnp.int32),
        pltpu.VMEM((NBUF, W, D, BLK), jnp.float32),
        pltpu.VMEM((NBUF, W, D, BLK), jnp.float32),
        pltpu.VMEM((BPW + PAD,), jnp.float32),
        pltpu.SemaphoreType.DMA,
        pltpu.SemaphoreType.DMA,
    ],
    compiler_params=pltpu.CompilerParams(
        needs_layout_passes=False, disable_bounds_checks=True
    ),
)
def _mf_sc(uid_hbm, iid_hbm, ut_hbm, it_hbm, out_hbm,
           uidx, iidx, ublk, iblk, outv, sem_u, sem_i):
    wid = lax.axis_index("s") * NC + lax.axis_index("c")
    base = wid * BPW
    pltpu.sync_copy(uid_hbm.at[pl.ds(base, BPW)], uidx.at[pl.ds(0, BPW)])
    pltpu.sync_copy(iid_hbm.at[pl.ds(base, BPW)], iidx.at[pl.ds(0, BPW)])

    lanes = lax.iota(jnp.int32, L)
    t_sel = lax.bitwise_and(lanes, jnp.int32(W - 1))
    lo_mask = lanes < W

    def fire(w):
        buf = lax.rem(w, NBUF)
        uv = uidx[pl.ds(w * W, L)]
        iv = iidx[pl.ds(w * W, L)]
        for t in range(W):
            ub = lax.shift_right_logical(uv[t], 7)
            ib = lax.shift_right_logical(iv[t], 7)
            pltpu.async_copy(
                ut_hbm.at[:, pl.ds(pl.multiple_of(ub * BLK, BLK), BLK)],
                ublk.at[buf, t], sem_u)
            pltpu.async_copy(
                it_hbm.at[:, pl.ds(pl.multiple_of(ib * BLK, BLK), BLK)],
                iblk.at[buf, t], sem_i)

    fire(jnp.int32(0))
    fire(jnp.int32(1))

    def wave(w, carry):
        @pl.when(w + 2 < NWAVE)
        def _():
            fire(w + 2)

        buf = lax.rem(w, NBUF)
        for t in range(W):
            pltpu.make_async_copy(
                ut_hbm.at[:, pl.ds(0, BLK)], ublk.at[buf, t], sem_u).wait()
            pltpu.make_async_copy(
                it_hbm.at[:, pl.ds(0, BLK)], iblk.at[buf, t], sem_i).wait()

        uv = uidx[pl.ds(w * W, L)]
        iv = iidx[pl.ds(w * W, L)]
        um = lax.bitwise_and(uv, jnp.int32(BLK - 1))
        im = lax.bitwise_and(iv, jnp.int32(BLK - 1))
        ub = ublk.at[buf]
        ib = iblk.at[buf]
        acc = jnp.zeros((L,), jnp.float32)
        for dd in range(D):
            dv = jnp.full((L,), dd, jnp.int32)
            ue = plsc.load_gather(ub, [t_sel, dv, um])
            ie = plsc.load_gather(ib, [t_sel, dv, im])
            acc = acc + ue * ie
        plsc.store_compressed(outv.at[pl.ds(w * W, L)], acc, mask=lo_mask)
        return carry

    lax.fori_loop(0, NWAVE, wave, 0, unroll=False)
    pltpu.sync_copy(outv.at[pl.ds(0, BPW)], out_hbm.at[pl.ds(base, BPW)])


def kernel(user_id, item_id, user_table, item_table):
    return _mf_sc(user_id.astype(jnp.int32), item_id.astype(jnp.int32),
                  user_table.T, item_table.T)
